# baseline (speedup 1.0000x reference)
import functools
import jax
import jax.numpy as jnp
from jax import lax
from jax.experimental import pallas as pl
from jax.experimental.pallas import tpu as pltpu
from jax.experimental.pallas import tpu_sc as plsc

N = 10000
E = 160000
IN_DIM = 128
HID = 64

NC = 2
NS = 16
NW = NC * NS
L = 16

CH = 128
KPT = 40
E_PAD = NW * KPT * CH
N_PAD = 10112
RPT = N_PAD // NS
DEGW = 16
NBUF = 8
ZROWS = RPT // 8

_MESH = plsc.VectorSubcoreMesh(core_axis_name="c", subcore_axis_name="s")
_SC_PARAMS = pltpu.CompilerParams(use_tc_tiling_on_sc=False)


def _fill(ref, rows, width, val):
    def body(i, _):
        for j in range(width // L):
            ref[i, pl.ds(j * L, L)] = jnp.full((L,), val, jnp.float32)
        return 0
    lax.fori_loop(0, rows, body, 0)


@functools.partial(
    pl.kernel,
    out_type=jax.ShapeDtypeStruct((NC, N_PAD, DEGW), jnp.float32),
    mesh=_MESH,
    compiler_params=_SC_PARAMS,
    scratch_types=[
        pltpu.VMEM((KPT, CH), jnp.int32),
        pltpu.VMEM((CH, DEGW), jnp.float32),
        pltpu.VMEM((RPT, DEGW), jnp.float32),
        pltpu.VMEM_SHARED((N_PAD, DEGW), jnp.float32),
    ],
)
def _sc_degree(dst_hbm, out_hbm, didx, ones_v, zbuf, acc):
    c = lax.axis_index("c")
    s = lax.axis_index("s")
    wid = c * NS + s
    _fill(ones_v, CH, DEGW, 1.0)
    _fill(zbuf, RPT, DEGW, 0.0)
    pltpu.sync_copy(zbuf, acc.at[pl.ds(s * RPT, RPT)])
    pltpu.sync_copy(dst_hbm.at[pl.ds(wid * KPT, KPT)], didx)
    plsc.subcore_barrier()

    def step(j, _):
        pltpu.sync_copy(ones_v, acc.at[didx.at[j]], add=True)
        return 0
    lax.fori_loop(0, KPT, step, 0)

    plsc.subcore_barrier()
    pltpu.sync_copy(acc.at[pl.ds(s * RPT, RPT)],
                    out_hbm.at[c, pl.ds(s * RPT, RPT)])


@functools.partial(
    pl.kernel,
    out_type=jax.ShapeDtypeStruct((NC, N_PAD, HID), jnp.float32),
    mesh=_MESH,
    compiler_params=_SC_PARAMS,
    scratch_types=[
        pltpu.VMEM((KPT, CH), jnp.int32),
        pltpu.VMEM((KPT, CH), jnp.int32),
        pltpu.VMEM((NBUF, CH, HID), jnp.float32),
        pltpu.VMEM((ZROWS, HID), jnp.float32),
        pltpu.VMEM_SHARED((N_PAD, HID), jnp.float32),
        pltpu.SemaphoreType.DMA,
        pltpu.SemaphoreType.DMA,
    ],
)
def _sc_prop(g_hbm, src_hbm, dst_hbm, out_hbm, sidx, didx, rows, zbuf, acc,
             gsem, ssem):
    c = lax.axis_index("c")
    s = lax.axis_index("s")
    wid = c * NS + s
    base = wid * KPT
    pltpu.sync_copy(src_hbm.at[pl.ds(base, KPT)], sidx)
    pltpu.sync_copy(dst_hbm.at[pl.ds(base, KPT)], didx)
    gd0 = [pltpu.async_copy(g_hbm.at[sidx.at[b]], rows.at[b], gsem)
           for b in range(NBUF)]
    _fill(zbuf, ZROWS, HID, 0.0)
    for r in range(8):
        pltpu.sync_copy(zbuf, acc.at[pl.ds(s * RPT + r * ZROWS, ZROWS)])
    plsc.subcore_barrier()

    sd0 = []
    for b in range(NBUF):
        gd0[b].wait()
        sd0.append(pltpu.async_copy(rows.at[b], acc.at[didx.at[b]],
                                    ssem, add=True))
    for d in sd0:
        d.wait()

    def group(g, _):
        j0 = g * NBUF
        gd = [pltpu.async_copy(g_hbm.at[sidx.at[j0 + b]], rows.at[b], gsem)
              for b in range(NBUF)]
        sd = []
        for b in range(NBUF):
            gd[b].wait()
            sd.append(pltpu.async_copy(rows.at[b], acc.at[didx.at[j0 + b]],
                                       ssem, add=True))
        for d in sd:
            d.wait()
        return 0
    lax.fori_loop(1, KPT // NBUF, group, 0)

    plsc.subcore_barrier()
    pltpu.sync_copy(acc.at[pl.ds(s * RPT, RPT)],
                    out_hbm.at[c, pl.ds(s * RPT, RPT)])


NP2 = N_PAD // 2


def _tc_pre_body(x2_ref, w1b_ref, degs_ref, sel_ref, rep_ref, dinv_ref, g1_ref):
    degsum = degs_ref[0] + degs_ref[1]
    d8 = jnp.dot(degsum, sel_ref[...], preferred_element_type=jnp.float32,
                 precision=lax.Precision.HIGHEST)
    dinv8 = lax.rsqrt(d8 + 1.0)
    dinv_p = jnp.dot(dinv8, rep_ref[...], preferred_element_type=jnp.float32,
                     precision=lax.Precision.HIGHEST).reshape(NP2, 128)
    dinv_ref[...] = dinv_p
    g1_ref[...] = dinv_p * jnp.dot(x2_ref[...], w1b_ref[...],
                                   preferred_element_type=jnp.float32)


def _tc_enc_body(s_ref, g_ref, dinv_ref, b_ref, w_ref, gn_ref):
    dinv = dinv_ref[...]
    p = dinv * (s_ref[0] + s_ref[1] + g_ref[...])
    h = jnp.maximum(p + b_ref[...], 0.0)
    gn_ref[...] = dinv * jnp.dot(h, w_ref[...], preferred_element_type=jnp.float32)


def _tc_emb_body(s_ref, g_ref, dinv_ref, b_ref, gn_ref):
    dinv = dinv_ref[...]
    emb = dinv * (s_ref[0] + s_ref[1] + g_ref[...]) + b_ref[...]
    gn_ref[...] = dinv * emb


def _tc_dec_body(s_ref, g_ref, dinv_ref, bd_ref, wd_ref, bs_ref, ws_ref,
                 gn_ref, hs_ref):
    dinv = dinv_ref[...]
    p = dinv * (s_ref[0] + s_ref[1] + g_ref[...])
    a = jnp.maximum(jnp.dot(p, wd_ref[...], preferred_element_type=jnp.float32)
                    + bd_ref[...], 0.0)
    gn_ref[...] = dinv * a
    hs_ref[...] = jnp.dot(p, ws_ref[...], preferred_element_type=jnp.float32) + bs_ref[...]


def _tc_out_body(s_ref, g_ref, dinv_ref, b_ref, w_ref, x_ref):
    dinv = dinv_ref[...]
    pa = dinv * (s_ref[0] + s_ref[1] + g_ref[...])
    x_ref[...] = jnp.dot(pa, w_ref[...], preferred_element_type=jnp.float32) + b_ref[...]


def _tc_gram_body(l_ref, r_ref, o_ref):
    o_ref[...] = jnp.dot(l_ref[...], r_ref[...],
                         preferred_element_type=jnp.float32)


def _blockdiag(w):
    a, b = w.shape
    z = jnp.zeros((2 * a, 2 * b), jnp.float32)
    return z.at[:a, :b].set(w).at[a:, b:].set(w)


def _full(shape_dtype_list, body):
    return pl.pallas_call(body, out_shape=shape_dtype_list)


_BM = 2048
_BN = 2560
_GM = (N + _BM - 1) // _BM
_GN = (N + _BN - 1) // _BN


def _gram(hs, hsT):
    return pl.pallas_call(
        _tc_gram_body,
        grid=(_GM, _GN),
        in_specs=[
            pl.BlockSpec((_BM, HID), lambda i, j: (i, 0)),
            pl.BlockSpec((HID, _BN), lambda i, j: (0, j)),
        ],
        out_specs=pl.BlockSpec((_BM, _BN), lambda i, j: (i, j)),
        out_shape=jax.ShapeDtypeStruct((N, N), jnp.float32),
    )(hs, hsT)


def kernel(x, edge_index, enc_W1, enc_b1, enc_W2, enc_b2,
           dec_W1, dec_b1, dec_W2, dec_b2, str_W, str_b):
    f32 = jnp.float32
    src = edge_index[0]
    dst = edge_index[1]
    pad = N + jnp.arange(E_PAD - E, dtype=jnp.int32) % (N_PAD - N)
    src_p = jnp.concatenate([src, pad]).reshape(NW * KPT, CH)
    dst_p = jnp.concatenate([dst, pad]).reshape(NW * KPT, CH)
    x2 = jnp.zeros((N_PAD, IN_DIM), f32).at[:N].set(x).reshape(NP2, 2 * IN_DIM)

    w1b = _blockdiag(enc_W1)
    w2b = _blockdiag(enc_W2)
    wd1b = _blockdiag(dec_W1)
    wd2b = _blockdiag(dec_W2)
    wsb = _blockdiag(str_W)
    b1b = jnp.concatenate([enc_b1, enc_b1]).reshape(1, 2 * HID)
    b2b = jnp.concatenate([enc_b2, enc_b2]).reshape(1, 2 * HID)
    bd1b = jnp.concatenate([dec_b1, dec_b1]).reshape(1, 2 * HID)
    bd2b = jnp.concatenate([dec_b2, dec_b2]).reshape(1, 2 * IN_DIM)
    bsb = jnp.concatenate([str_b, str_b]).reshape(1, 2 * HID)
    sel = jnp.zeros((128, 8), f32).at[16 * jnp.arange(8), jnp.arange(8)].set(1.0)
    rep = jnp.zeros((8, 512), f32)
    rep = rep.at[jnp.repeat(jnp.arange(8), 64),
                 jnp.arange(512)].set(1.0)

    degs = _sc_degree(dst_p)
    degs_p = degs.reshape(NC, N_PAD * DEGW // 128, 128)

    sd = lambda shape: jax.ShapeDtypeStruct(shape, f32)
    sp = lambda a: a.reshape(NC, NP2, 128)
    dinv, g1 = _full([sd((NP2, 128)), sd((NP2, 128))], _tc_pre_body)(
        x2, w1b, degs_p, sel, rep)
    s1 = _sc_prop(g1.reshape(N_PAD, HID), src_p, dst_p)
    g2 = _full(sd((NP2, 128)), _tc_enc_body)(sp(s1), g1, dinv, b1b, w2b)
    s2 = _sc_prop(g2.reshape(N_PAD, HID), src_p, dst_p)
    g3 = _full(sd((NP2, 128)), _tc_emb_body)(sp(s2), g2, dinv, b2b)
    s3 = _sc_prop(g3.reshape(N_PAD, HID), src_p, dst_p)
    g4, hs_p = _full([sd((NP2, 128)), sd((NP2, 128))], _tc_dec_body)(
        sp(s3), g3, dinv, bd1b, wd1b, bsb, wsb)
    s4 = _sc_prop(g4.reshape(N_PAD, HID), src_p, dst_p)
    x2o = _full(sd((NP2, 2 * IN_DIM)), _tc_out_body)(
        sp(s4), g4, dinv, bd2b, wd2b)

    x_ = x2o.reshape(N_PAD, IN_DIM)[:N]
    hs_n = hs_p.reshape(N_PAD, HID)[:N]
    s_ = _gram(hs_n, hs_n.T)
    return (x_, s_)


# speedup vs baseline: 1.0600x; 1.0600x over previous
import functools
import jax
import jax.numpy as jnp
from jax import lax
from jax.experimental import pallas as pl
from jax.experimental.pallas import tpu as pltpu
from jax.experimental.pallas import tpu_sc as plsc

N = 10000
E = 160000
IN_DIM = 128
HID = 64

NC = 2
NS = 16
NW = NC * NS
L = 16

CH = 128
KPT = 40
E_PAD = NW * KPT * CH
N_PAD = 10112
RPT = N_PAD // NS
DEGW = 16
NBUF = 8
ZROWS = RPT // 8

_MESH = plsc.VectorSubcoreMesh(core_axis_name="c", subcore_axis_name="s")
_SC_PARAMS = pltpu.CompilerParams(use_tc_tiling_on_sc=False)


def _fill(ref, rows, width, val):
    def body(i, _):
        for j in range(width // L):
            ref[i, pl.ds(j * L, L)] = jnp.full((L,), val, jnp.float32)
        return 0
    lax.fori_loop(0, rows, body, 0)


@functools.partial(
    pl.kernel,
    out_type=jax.ShapeDtypeStruct((NC, N_PAD, DEGW), jnp.float32),
    mesh=_MESH,
    compiler_params=_SC_PARAMS,
    scratch_types=[
        pltpu.VMEM((KPT, CH), jnp.int32),
        pltpu.VMEM((CH, DEGW), jnp.float32),
        pltpu.VMEM((RPT, DEGW), jnp.float32),
        pltpu.VMEM_SHARED((N_PAD, DEGW), jnp.float32),
    ],
)
def _sc_degree(dst_hbm, out_hbm, didx, ones_v, zbuf, acc):
    c = lax.axis_index("c")
    s = lax.axis_index("s")
    wid = c * NS + s
    _fill(ones_v, CH, DEGW, 1.0)
    _fill(zbuf, RPT, DEGW, 0.0)
    pltpu.sync_copy(zbuf, acc.at[pl.ds(s * RPT, RPT)])
    pltpu.sync_copy(dst_hbm.at[pl.ds(wid * KPT, KPT)], didx)
    plsc.subcore_barrier()

    def step(j, _):
        pltpu.sync_copy(ones_v, acc.at[didx.at[j]], add=True)
        return 0
    lax.fori_loop(0, KPT, step, 0)

    plsc.subcore_barrier()
    pltpu.sync_copy(acc.at[pl.ds(s * RPT, RPT)],
                    out_hbm.at[c, pl.ds(s * RPT, RPT)])


@functools.partial(
    pl.kernel,
    out_type=jax.ShapeDtypeStruct((NC, N_PAD, HID), jnp.float32),
    mesh=_MESH,
    compiler_params=_SC_PARAMS,
    scratch_types=[
        pltpu.VMEM((KPT, CH), jnp.int32),
        pltpu.VMEM((KPT, CH), jnp.int32),
        pltpu.VMEM((NBUF, CH, HID), jnp.float32),
        pltpu.VMEM((ZROWS, HID), jnp.float32),
        pltpu.VMEM_SHARED((N_PAD, HID), jnp.float32),
        pltpu.SemaphoreType.DMA,
        pltpu.SemaphoreType.DMA,
    ],
)
def _sc_prop(g_hbm, src_hbm, dst_hbm, out_hbm, sidx, didx, rows, zbuf, acc,
             gsem, ssem):
    c = lax.axis_index("c")
    s = lax.axis_index("s")
    wid = c * NS + s
    base = wid * KPT
    pltpu.sync_copy(src_hbm.at[pl.ds(base, KPT)], sidx)
    pltpu.sync_copy(dst_hbm.at[pl.ds(base, KPT)], didx)
    gd0 = [pltpu.async_copy(g_hbm.at[sidx.at[b]], rows.at[b], gsem)
           for b in range(NBUF)]
    _fill(zbuf, ZROWS, HID, 0.0)
    for r in range(8):
        pltpu.sync_copy(zbuf, acc.at[pl.ds(s * RPT + r * ZROWS, ZROWS)])
    plsc.subcore_barrier()

    sd0 = []
    for b in range(NBUF):
        gd0[b].wait()
        sd0.append(pltpu.async_copy(rows.at[b], acc.at[didx.at[b]],
                                    ssem, add=True))
    for d in sd0:
        d.wait()

    def group(g, _):
        j0 = g * NBUF
        gd = [pltpu.async_copy(g_hbm.at[sidx.at[j0 + b]], rows.at[b], gsem)
              for b in range(NBUF)]
        sd = []
        for b in range(NBUF):
            gd[b].wait()
            sd.append(pltpu.async_copy(rows.at[b], acc.at[didx.at[j0 + b]],
                                       ssem, add=True))
        for d in sd:
            d.wait()
        return 0
    lax.fori_loop(1, KPT // NBUF, group, 0)

    plsc.subcore_barrier()
    pltpu.sync_copy(acc.at[pl.ds(s * RPT, RPT)],
                    out_hbm.at[c, pl.ds(s * RPT, RPT)])


NP2 = N_PAD // 2


def _tc_pre_body(x2_ref, w1b_ref, degs_ref, sel_ref, rep_ref, dinv_ref, g1_ref):
    degsum = degs_ref[0] + degs_ref[1]
    d8 = jnp.dot(degsum, sel_ref[...], preferred_element_type=jnp.float32,
                 precision=lax.Precision.HIGHEST)
    dinv8 = lax.rsqrt(d8 + 1.0)
    dinv_p = jnp.dot(dinv8, rep_ref[...], preferred_element_type=jnp.float32,
                     precision=lax.Precision.HIGHEST).reshape(NP2, 128)
    dinv_ref[...] = dinv_p
    g1_ref[...] = dinv_p * jnp.dot(x2_ref[...], w1b_ref[...],
                                   preferred_element_type=jnp.float32)


def _tc_enc_body(s_ref, g_ref, dinv_ref, b_ref, w_ref, gn_ref):
    dinv = dinv_ref[...]
    p = dinv * (s_ref[0] + s_ref[1] + g_ref[...])
    h = jnp.maximum(p + b_ref[...], 0.0)
    gn_ref[...] = dinv * jnp.dot(h, w_ref[...], preferred_element_type=jnp.float32)


def _tc_emb_body(s_ref, g_ref, dinv_ref, b_ref, gn_ref):
    dinv = dinv_ref[...]
    emb = dinv * (s_ref[0] + s_ref[1] + g_ref[...]) + b_ref[...]
    gn_ref[...] = dinv * emb


def _tc_dec_body(s_ref, g_ref, dinv_ref, bd_ref, wd_ref, bs_ref, ws_ref,
                 gn_ref, hs_ref):
    dinv = dinv_ref[...]
    p = dinv * (s_ref[0] + s_ref[1] + g_ref[...])
    a = jnp.maximum(jnp.dot(p, wd_ref[...], preferred_element_type=jnp.float32)
                    + bd_ref[...], 0.0)
    gn_ref[...] = dinv * a
    hs_ref[...] = jnp.dot(p, ws_ref[...], preferred_element_type=jnp.float32) + bs_ref[...]


def _tc_out_body(s_ref, g_ref, dinv_ref, b_ref, w_ref, x_ref):
    dinv = dinv_ref[...]
    pa = dinv * (s_ref[0] + s_ref[1] + g_ref[...])
    x_ref[...] = jnp.dot(pa, w_ref[...], preferred_element_type=jnp.float32) + b_ref[...]


def _tc_gram_body(l_ref, r_ref, o_ref):
    o_ref[...] = jnp.dot(l_ref[...], r_ref[...],
                         preferred_element_type=jnp.float32)


def _blockdiag(w):
    a, b = w.shape
    z = jnp.zeros((a, b), jnp.float32)
    return jnp.concatenate([jnp.concatenate([w, z], axis=1),
                            jnp.concatenate([z, w], axis=1)], axis=0)


def _full(shape_dtype_list, body):
    return pl.pallas_call(body, out_shape=shape_dtype_list)


_BM = 2048
_BN = 2560
_GM = (N + _BM - 1) // _BM
_GN = (N + _BN - 1) // _BN


def _gram(hs, hsT):
    return pl.pallas_call(
        _tc_gram_body,
        grid=(_GM, _GN),
        in_specs=[
            pl.BlockSpec((_BM, HID), lambda i, j: (i, 0)),
            pl.BlockSpec((HID, _BN), lambda i, j: (0, j)),
        ],
        out_specs=pl.BlockSpec((_BM, _BN), lambda i, j: (i, j)),
        out_shape=jax.ShapeDtypeStruct((N, N), jnp.float32),
    )(hs, hsT)


def kernel(x, edge_index, enc_W1, enc_b1, enc_W2, enc_b2,
           dec_W1, dec_b1, dec_W2, dec_b2, str_W, str_b):
    f32 = jnp.float32
    src = edge_index[0]
    dst = edge_index[1]
    pad = N + jnp.arange(E_PAD - E, dtype=jnp.int32) % (N_PAD - N)
    src_p = jnp.concatenate([src, pad]).reshape(NW * KPT, CH)
    dst_p = jnp.concatenate([dst, pad]).reshape(NW * KPT, CH)
    x2 = jnp.zeros((N_PAD, IN_DIM), f32).at[:N].set(x).reshape(NP2, 2 * IN_DIM)

    w1b = _blockdiag(enc_W1)
    w2b = _blockdiag(enc_W2)
    wd1b = _blockdiag(dec_W1)
    wd2b = _blockdiag(dec_W2)
    wsb = _blockdiag(str_W)
    b1b = jnp.concatenate([enc_b1, enc_b1]).reshape(1, 2 * HID)
    b2b = jnp.concatenate([enc_b2, enc_b2]).reshape(1, 2 * HID)
    bd1b = jnp.concatenate([dec_b1, dec_b1]).reshape(1, 2 * HID)
    bd2b = jnp.concatenate([dec_b2, dec_b2]).reshape(1, 2 * IN_DIM)
    bsb = jnp.concatenate([str_b, str_b]).reshape(1, 2 * HID)
    sel = (lax.broadcasted_iota(jnp.int32, (128, 8), 0)
           == 16 * lax.broadcasted_iota(jnp.int32, (128, 8), 1)).astype(f32)
    rep = (lax.broadcasted_iota(jnp.int32, (8, 512), 0)
           == lax.broadcasted_iota(jnp.int32, (8, 512), 1) // 64).astype(f32)

    degs = _sc_degree(dst_p)
    degs_p = degs.reshape(NC, N_PAD * DEGW // 128, 128)

    sd = lambda shape: jax.ShapeDtypeStruct(shape, f32)
    sp = lambda a: a.reshape(NC, NP2, 128)
    dinv, g1 = _full([sd((NP2, 128)), sd((NP2, 128))], _tc_pre_body)(
        x2, w1b, degs_p, sel, rep)
    s1 = _sc_prop(g1.reshape(N_PAD, HID), src_p, dst_p)
    g2 = _full(sd((NP2, 128)), _tc_enc_body)(sp(s1), g1, dinv, b1b, w2b)
    s2 = _sc_prop(g2.reshape(N_PAD, HID), src_p, dst_p)
    g3 = _full(sd((NP2, 128)), _tc_emb_body)(sp(s2), g2, dinv, b2b)
    s3 = _sc_prop(g3.reshape(N_PAD, HID), src_p, dst_p)
    g4, hs_p = _full([sd((NP2, 128)), sd((NP2, 128))], _tc_dec_body)(
        sp(s3), g3, dinv, bd1b, wd1b, bsb, wsb)
    s4 = _sc_prop(g4.reshape(N_PAD, HID), src_p, dst_p)
    x2o = _full(sd((NP2, 2 * IN_DIM)), _tc_out_body)(
        sp(s4), g4, dinv, bd2b, wd2b)

    x_ = x2o.reshape(N_PAD, IN_DIM)[:N]
    hs_n = hs_p.reshape(N_PAD, HID)[:N]
    s_ = _gram(hs_n, hs_n.T)
    return (x_, s_)


# speedup vs baseline: 1.1131x; 1.0501x over previous
import functools
import jax
import jax.numpy as jnp
from jax import lax
from jax.experimental import pallas as pl
from jax.experimental.pallas import tpu as pltpu
from jax.experimental.pallas import tpu_sc as plsc

N = 10000
E = 160000
IN_DIM = 128
HID = 64

NC = 2
NS = 16
NW = NC * NS
L = 16

CH = 128
KPT = 40
E_PAD = NW * KPT * CH
N_PAD = 10112
RPT = N_PAD // NS
DEGW = 16
GRP = 4
NBUF = 2 * GRP
NGRP = KPT // GRP
ZROWS = RPT // 8

_MESH = plsc.VectorSubcoreMesh(core_axis_name="c", subcore_axis_name="s")
_SC_PARAMS = pltpu.CompilerParams(use_tc_tiling_on_sc=False)


def _fill(ref, rows, width, val):
    def body(i, _):
        for j in range(width // L):
            ref[i, pl.ds(j * L, L)] = jnp.full((L,), val, jnp.float32)
        return 0
    lax.fori_loop(0, rows, body, 0)


@functools.partial(
    pl.kernel,
    out_type=jax.ShapeDtypeStruct((NC, N_PAD, DEGW), jnp.float32),
    mesh=_MESH,
    compiler_params=_SC_PARAMS,
    scratch_types=[
        pltpu.VMEM((KPT, CH), jnp.int32),
        pltpu.VMEM((CH, DEGW), jnp.float32),
        pltpu.VMEM((RPT, DEGW), jnp.float32),
        pltpu.VMEM_SHARED((N_PAD, DEGW), jnp.float32),
    ],
)
def _sc_degree(dst_hbm, out_hbm, didx, ones_v, zbuf, acc):
    c = lax.axis_index("c")
    s = lax.axis_index("s")
    wid = c * NS + s
    _fill(ones_v, CH, DEGW, 1.0)
    _fill(zbuf, RPT, DEGW, 0.0)
    pltpu.sync_copy(zbuf, acc.at[pl.ds(s * RPT, RPT)])
    pltpu.sync_copy(dst_hbm.at[pl.ds(wid * KPT, KPT)], didx)
    plsc.subcore_barrier()

    def step(j, _):
        pltpu.sync_copy(ones_v, acc.at[didx.at[j]], add=True)
        return 0
    lax.fori_loop(0, KPT, step, 0)

    plsc.subcore_barrier()
    pltpu.sync_copy(acc.at[pl.ds(s * RPT, RPT)],
                    out_hbm.at[c, pl.ds(s * RPT, RPT)])


@functools.partial(
    pl.kernel,
    out_type=jax.ShapeDtypeStruct((NC, N_PAD, HID), jnp.float32),
    mesh=_MESH,
    compiler_params=_SC_PARAMS,
    scratch_types=[
        pltpu.VMEM((KPT, CH), jnp.int32),
        pltpu.VMEM((KPT, CH), jnp.int32),
        pltpu.VMEM((NBUF, CH, HID), jnp.float32),
        pltpu.VMEM((ZROWS, HID), jnp.float32),
        pltpu.VMEM_SHARED((N_PAD, HID), jnp.float32),
        pltpu.SemaphoreType.DMA,
        pltpu.SemaphoreType.DMA,
    ],
)
def _sc_prop(g_hbm, src_hbm, dst_hbm, out_hbm, sidx, didx, rows, zbuf, acc,
             gsem, ssem):
    c = lax.axis_index("c")
    s = lax.axis_index("s")
    wid = c * NS + s
    base = wid * KPT
    pltpu.sync_copy(src_hbm.at[pl.ds(base, KPT)], sidx)
    pltpu.sync_copy(dst_hbm.at[pl.ds(base, KPT)], didx)
    for b in range(GRP):
        pltpu.async_copy(g_hbm.at[sidx.at[b]], rows.at[b], gsem)
    _fill(zbuf, ZROWS, HID, 0.0)
    for r in range(8):
        pltpu.sync_copy(zbuf, acc.at[pl.ds(s * RPT + r * ZROWS, ZROWS)])
    plsc.subcore_barrier()

    def gwait(b):
        pltpu.make_async_copy(g_hbm.at[sidx.at[0]], rows.at[b], gsem).wait()

    def swait(b):
        pltpu.make_async_copy(rows.at[b], acc.at[didx.at[0]], ssem).wait()

    def step(t, _):
        def phase(cur, nxt):
            @pl.when(t > 0)
            def _drain():
                for b in range(GRP):
                    swait(nxt + b)

            @pl.when(t < NGRP - 1)
            def _prefetch():
                for b in range(GRP):
                    pltpu.async_copy(g_hbm.at[sidx.at[(t + 1) * GRP + b]],
                                     rows.at[nxt + b], gsem)
            for b in range(GRP):
                gwait(cur + b)
                pltpu.async_copy(rows.at[cur + b],
                                 acc.at[didx.at[t * GRP + b]], ssem, add=True)

        even = lax.rem(t, 2) == 0

        @pl.when(even)
        def _a():
            phase(0, GRP)

        @pl.when(jnp.logical_not(even))
        def _b():
            phase(GRP, 0)
        return 0
    lax.fori_loop(0, NGRP, step, 0)
    for b in range(GRP):
        swait(GRP + b)

    plsc.subcore_barrier()
    pltpu.sync_copy(acc.at[pl.ds(s * RPT, RPT)],
                    out_hbm.at[c, pl.ds(s * RPT, RPT)])


NP2 = N_PAD // 2


def _tc_pre_body(x2_ref, w1b_ref, degs_ref, sel_ref, rep_ref, dinv_ref, g1_ref):
    degsum = degs_ref[0] + degs_ref[1]
    d8 = jnp.dot(degsum, sel_ref[...], preferred_element_type=jnp.float32,
                 precision=lax.Precision.HIGHEST)
    dinv8 = lax.rsqrt(d8 + 1.0)
    dinv_p = jnp.dot(dinv8, rep_ref[...], preferred_element_type=jnp.float32,
                     precision=lax.Precision.HIGHEST).reshape(NP2, 128)
    dinv_ref[...] = dinv_p
    g1_ref[...] = dinv_p * jnp.dot(x2_ref[...], w1b_ref[...],
                                   preferred_element_type=jnp.float32)


def _tc_enc_body(s_ref, g_ref, dinv_ref, b_ref, w_ref, gn_ref):
    dinv = dinv_ref[...]
    p = dinv * (s_ref[0] + s_ref[1] + g_ref[...])
    h = jnp.maximum(p + b_ref[...], 0.0)
    gn_ref[...] = dinv * jnp.dot(h, w_ref[...], preferred_element_type=jnp.float32)


def _tc_emb_body(s_ref, g_ref, dinv_ref, b_ref, gn_ref):
    dinv = dinv_ref[...]
    emb = dinv * (s_ref[0] + s_ref[1] + g_ref[...]) + b_ref[...]
    gn_ref[...] = dinv * emb


def _tc_dec_body(s_ref, g_ref, dinv_ref, bd_ref, wd_ref, bs_ref, ws_ref,
                 gn_ref, hs_ref):
    dinv = dinv_ref[...]
    p = dinv * (s_ref[0] + s_ref[1] + g_ref[...])
    a = jnp.maximum(jnp.dot(p, wd_ref[...], preferred_element_type=jnp.float32)
                    + bd_ref[...], 0.0)
    gn_ref[...] = dinv * a
    hs_ref[...] = jnp.dot(p, ws_ref[...], preferred_element_type=jnp.float32) + bs_ref[...]


def _tc_out_body(s_ref, g_ref, dinv_ref, b_ref, w_ref, x_ref):
    dinv = dinv_ref[...]
    pa = dinv * (s_ref[0] + s_ref[1] + g_ref[...])
    x_ref[...] = jnp.dot(pa, w_ref[...], preferred_element_type=jnp.float32) + b_ref[...]


def _tc_gram_body(l_ref, r_ref, o_ref):
    o_ref[...] = jnp.dot(l_ref[...], r_ref[...],
                         preferred_element_type=jnp.float32)


def _blockdiag(w):
    a, b = w.shape
    z = jnp.zeros((a, b), jnp.float32)
    return jnp.concatenate([jnp.concatenate([w, z], axis=1),
                            jnp.concatenate([z, w], axis=1)], axis=0)


def _full(shape_dtype_list, body):
    return pl.pallas_call(body, out_shape=shape_dtype_list)


_BM = 2048
_BN = 2560
_GM = (N + _BM - 1) // _BM
_GN = (N + _BN - 1) // _BN


def _gram(hs, hsT):
    return pl.pallas_call(
        _tc_gram_body,
        grid=(_GM, _GN),
        in_specs=[
            pl.BlockSpec((_BM, HID), lambda i, j: (i, 0)),
            pl.BlockSpec((HID, _BN), lambda i, j: (0, j)),
        ],
        out_specs=pl.BlockSpec((_BM, _BN), lambda i, j: (i, j)),
        out_shape=jax.ShapeDtypeStruct((N, N), jnp.float32),
    )(hs, hsT)


def kernel(x, edge_index, enc_W1, enc_b1, enc_W2, enc_b2,
           dec_W1, dec_b1, dec_W2, dec_b2, str_W, str_b):
    f32 = jnp.float32
    src = edge_index[0]
    dst = edge_index[1]
    pad = N + jnp.arange(E_PAD - E, dtype=jnp.int32) % (N_PAD - N)
    src_p = jnp.concatenate([src, pad]).reshape(NW * KPT, CH)
    dst_p = jnp.concatenate([dst, pad]).reshape(NW * KPT, CH)
    x2 = jnp.zeros((N_PAD, IN_DIM), f32).at[:N].set(x).reshape(NP2, 2 * IN_DIM)

    w1b = _blockdiag(enc_W1)
    w2b = _blockdiag(enc_W2)
    wd1b = _blockdiag(dec_W1)
    wd2b = _blockdiag(dec_W2)
    wsb = _blockdiag(str_W)
    b1b = jnp.concatenate([enc_b1, enc_b1]).reshape(1, 2 * HID)
    b2b = jnp.concatenate([enc_b2, enc_b2]).reshape(1, 2 * HID)
    bd1b = jnp.concatenate([dec_b1, dec_b1]).reshape(1, 2 * HID)
    bd2b = jnp.concatenate([dec_b2, dec_b2]).reshape(1, 2 * IN_DIM)
    bsb = jnp.concatenate([str_b, str_b]).reshape(1, 2 * HID)
    sel = (lax.broadcasted_iota(jnp.int32, (128, 8), 0)
           == 16 * lax.broadcasted_iota(jnp.int32, (128, 8), 1)).astype(f32)
    rep = (lax.broadcasted_iota(jnp.int32, (8, 512), 0)
           == lax.broadcasted_iota(jnp.int32, (8, 512), 1) // 64).astype(f32)

    degs = _sc_degree(dst_p)
    degs_p = degs.reshape(NC, N_PAD * DEGW // 128, 128)

    sd = lambda shape: jax.ShapeDtypeStruct(shape, f32)
    sp = lambda a: a.reshape(NC, NP2, 128)
    dinv, g1 = _full([sd((NP2, 128)), sd((NP2, 128))], _tc_pre_body)(
        x2, w1b, degs_p, sel, rep)
    s1 = _sc_prop(g1.reshape(N_PAD, HID), src_p, dst_p)
    g2 = _full(sd((NP2, 128)), _tc_enc_body)(sp(s1), g1, dinv, b1b, w2b)
    s2 = _sc_prop(g2.reshape(N_PAD, HID), src_p, dst_p)
    g3 = _full(sd((NP2, 128)), _tc_emb_body)(sp(s2), g2, dinv, b2b)
    s3 = _sc_prop(g3.reshape(N_PAD, HID), src_p, dst_p)
    g4, hs_p = _full([sd((NP2, 128)), sd((NP2, 128))], _tc_dec_body)(
        sp(s3), g3, dinv, bd1b, wd1b, bsb, wsb)
    s4 = _sc_prop(g4.reshape(N_PAD, HID), src_p, dst_p)
    x2o = _full(sd((NP2, 2 * IN_DIM)), _tc_out_body)(
        sp(s4), g4, dinv, bd2b, wd2b)

    x_ = x2o.reshape(N_PAD, IN_DIM)[:N]
    hs_n = hs_p.reshape(N_PAD, HID)[:N]
    s_ = _gram(hs_n, hs_n.T)
    return (x_, s_)


# speedup vs baseline: 1.1190x; 1.0053x over previous
import functools
import jax
import jax.numpy as jnp
from jax import lax
from jax.experimental import pallas as pl
from jax.experimental.pallas import tpu as pltpu
from jax.experimental.pallas import tpu_sc as plsc

N = 10000
E = 160000
IN_DIM = 128
HID = 64

NC = 2
NS = 16
NW = NC * NS
L = 16

CH = 128
KPT = 40
E_PAD = NW * KPT * CH
N_PAD = 10112
RPT = N_PAD // NS
DEGW = 16
GRP = 4
NBUF = 2 * GRP
NGRP = KPT // GRP
ZROWS = RPT // 8

_MESH = plsc.VectorSubcoreMesh(core_axis_name="c", subcore_axis_name="s")
_SC_PARAMS = pltpu.CompilerParams(use_tc_tiling_on_sc=False)


def _fill(ref, rows, width, val):
    def body(i, _):
        for j in range(width // L):
            ref[i, pl.ds(j * L, L)] = jnp.full((L,), val, jnp.float32)
        return 0
    lax.fori_loop(0, rows, body, 0)


@functools.partial(
    pl.kernel,
    out_type=jax.ShapeDtypeStruct((NC, N_PAD, DEGW), jnp.float32),
    mesh=_MESH,
    compiler_params=_SC_PARAMS,
    scratch_types=[
        pltpu.VMEM((KPT, CH), jnp.int32),
        pltpu.VMEM((CH, DEGW), jnp.float32),
        pltpu.VMEM((RPT, DEGW), jnp.float32),
        pltpu.VMEM_SHARED((N_PAD, DEGW), jnp.float32),
        pltpu.SemaphoreType.DMA,
    ],
)
def _sc_degree(dst_hbm, out_hbm, didx, ones_v, zbuf, acc, sem):
    c = lax.axis_index("c")
    s = lax.axis_index("s")
    wid = c * NS + s
    _fill(ones_v, CH, DEGW, 1.0)
    _fill(zbuf, RPT, DEGW, 0.0)
    pltpu.sync_copy(zbuf, acc.at[pl.ds(s * RPT, RPT)])
    pltpu.sync_copy(dst_hbm.at[pl.ds(wid * KPT, KPT)], didx)
    plsc.subcore_barrier()

    def step(j, _):
        for b in range(GRP):
            pltpu.async_copy(ones_v, acc.at[didx.at[j * GRP + b]], sem,
                             add=True)
        for b in range(GRP):
            pltpu.make_async_copy(ones_v, acc.at[didx.at[0]], sem).wait()
        return 0
    lax.fori_loop(0, KPT // GRP, step, 0)

    plsc.subcore_barrier()
    pltpu.sync_copy(acc.at[pl.ds(s * RPT, RPT)],
                    out_hbm.at[c, pl.ds(s * RPT, RPT)])


@functools.partial(
    pl.kernel,
    out_type=jax.ShapeDtypeStruct((NC, N_PAD, HID), jnp.float32),
    mesh=_MESH,
    compiler_params=_SC_PARAMS,
    scratch_types=[
        pltpu.VMEM((KPT, CH), jnp.int32),
        pltpu.VMEM((KPT, CH), jnp.int32),
        pltpu.VMEM((NBUF, CH, HID), jnp.float32),
        pltpu.VMEM((ZROWS, HID), jnp.float32),
        pltpu.VMEM_SHARED((N_PAD, HID), jnp.float32),
        pltpu.SemaphoreType.DMA,
        pltpu.SemaphoreType.DMA,
    ],
)
def _sc_prop(g_hbm, src_hbm, dst_hbm, out_hbm, sidx, didx, rows, zbuf, acc,
             gsem, ssem):
    c = lax.axis_index("c")
    s = lax.axis_index("s")
    wid = c * NS + s
    base = wid * KPT
    pltpu.sync_copy(src_hbm.at[pl.ds(base, KPT)], sidx)
    pltpu.sync_copy(dst_hbm.at[pl.ds(base, KPT)], didx)
    for b in range(GRP):
        pltpu.async_copy(g_hbm.at[sidx.at[b]], rows.at[b], gsem)
    _fill(zbuf, ZROWS, HID, 0.0)
    for r in range(8):
        pltpu.sync_copy(zbuf, acc.at[pl.ds(s * RPT + r * ZROWS, ZROWS)])
    plsc.subcore_barrier()

    def gwait(b):
        pltpu.make_async_copy(g_hbm.at[sidx.at[0]], rows.at[b], gsem).wait()

    def swait(b):
        pltpu.make_async_copy(rows.at[b], acc.at[didx.at[0]], ssem).wait()

    def step(t, _):
        def phase(cur, nxt):
            @pl.when(t > 0)
            def _drain():
                for b in range(GRP):
                    swait(nxt + b)

            @pl.when(t < NGRP - 1)
            def _prefetch():
                for b in range(GRP):
                    pltpu.async_copy(g_hbm.at[sidx.at[(t + 1) * GRP + b]],
                                     rows.at[nxt + b], gsem)
            for b in range(GRP):
                gwait(cur + b)
                pltpu.async_copy(rows.at[cur + b],
                                 acc.at[didx.at[t * GRP + b]], ssem, add=True)

        even = lax.rem(t, 2) == 0

        @pl.when(even)
        def _a():
            phase(0, GRP)

        @pl.when(jnp.logical_not(even))
        def _b():
            phase(GRP, 0)
        return 0
    lax.fori_loop(0, NGRP, step, 0)
    for b in range(GRP):
        swait(GRP + b)

    plsc.subcore_barrier()
    pltpu.sync_copy(acc.at[pl.ds(s * RPT, RPT)],
                    out_hbm.at[c, pl.ds(s * RPT, RPT)])


NP2 = N_PAD // 2


def _tc_pre_body(x2_ref, w1b_ref, degs_ref, sel_ref, rep_ref, dinv_ref, g1_ref):
    degsum = degs_ref[0] + degs_ref[1]
    d8 = jnp.dot(degsum, sel_ref[...], preferred_element_type=jnp.float32,
                 precision=lax.Precision.HIGHEST)
    dinv8 = lax.rsqrt(d8 + 1.0)
    dinv_p = jnp.dot(dinv8, rep_ref[...], preferred_element_type=jnp.float32,
                     precision=lax.Precision.HIGHEST).reshape(NP2, 128)
    dinv_ref[...] = dinv_p
    g1_ref[...] = dinv_p * jnp.dot(x2_ref[...], w1b_ref[...],
                                   preferred_element_type=jnp.float32)


def _tc_enc_body(s_ref, g_ref, dinv_ref, b_ref, w_ref, gn_ref):
    dinv = dinv_ref[...]
    p = dinv * (s_ref[0] + s_ref[1] + g_ref[...])
    h = jnp.maximum(p + b_ref[...], 0.0)
    gn_ref[...] = dinv * jnp.dot(h, w_ref[...], preferred_element_type=jnp.float32)


def _tc_emb_body(s_ref, g_ref, dinv_ref, b_ref, gn_ref):
    dinv = dinv_ref[...]
    emb = dinv * (s_ref[0] + s_ref[1] + g_ref[...]) + b_ref[...]
    gn_ref[...] = dinv * emb


def _tc_dec_body(s_ref, g_ref, dinv_ref, bd_ref, wd_ref, bs_ref, ws_ref,
                 gn_ref, hs_ref):
    dinv = dinv_ref[...]
    p = dinv * (s_ref[0] + s_ref[1] + g_ref[...])
    a = jnp.maximum(jnp.dot(p, wd_ref[...], preferred_element_type=jnp.float32)
                    + bd_ref[...], 0.0)
    gn_ref[...] = dinv * a
    hs_ref[...] = jnp.dot(p, ws_ref[...], preferred_element_type=jnp.float32) + bs_ref[...]


def _tc_out_body(s_ref, g_ref, dinv_ref, b_ref, w_ref, x_ref):
    dinv = dinv_ref[...]
    pa = dinv * (s_ref[0] + s_ref[1] + g_ref[...])
    x_ref[...] = jnp.dot(pa, w_ref[...], preferred_element_type=jnp.float32) + b_ref[...]


def _tc_gram_body(l_ref, r_ref, o_ref):
    o_ref[...] = jnp.dot(l_ref[...], r_ref[...],
                         preferred_element_type=jnp.float32)


def _blockdiag(w):
    a, b = w.shape
    z = jnp.zeros((a, b), jnp.float32)
    return jnp.concatenate([jnp.concatenate([w, z], axis=1),
                            jnp.concatenate([z, w], axis=1)], axis=0)


def _full(shape_dtype_list, body):
    return pl.pallas_call(body, out_shape=shape_dtype_list)


_BM = 2048
_BN = 2560
_GM = (N + _BM - 1) // _BM
_GN = (N + _BN - 1) // _BN


def _gram(hs, hsT):
    return pl.pallas_call(
        _tc_gram_body,
        grid=(_GM, _GN),
        in_specs=[
            pl.BlockSpec((_BM, HID), lambda i, j: (i, 0)),
            pl.BlockSpec((HID, _BN), lambda i, j: (0, j)),
        ],
        out_specs=pl.BlockSpec((_BM, _BN), lambda i, j: (i, j)),
        out_shape=jax.ShapeDtypeStruct((N, N), jnp.float32),
    )(hs, hsT)


def kernel(x, edge_index, enc_W1, enc_b1, enc_W2, enc_b2,
           dec_W1, dec_b1, dec_W2, dec_b2, str_W, str_b):
    f32 = jnp.float32
    src = edge_index[0]
    dst = edge_index[1]
    pad = N + jnp.arange(E_PAD - E, dtype=jnp.int32) % (N_PAD - N)
    src_p = jnp.concatenate([src, pad]).reshape(NW * KPT, CH)
    dst_p = jnp.concatenate([dst, pad]).reshape(NW * KPT, CH)
    x2 = jnp.zeros((N_PAD, IN_DIM), f32).at[:N].set(x).reshape(NP2, 2 * IN_DIM)

    w1b = _blockdiag(enc_W1)
    w2b = _blockdiag(enc_W2)
    wd1b = _blockdiag(dec_W1)
    wd2b = _blockdiag(dec_W2)
    wsb = _blockdiag(str_W)
    b1b = jnp.concatenate([enc_b1, enc_b1]).reshape(1, 2 * HID)
    b2b = jnp.concatenate([enc_b2, enc_b2]).reshape(1, 2 * HID)
    bd1b = jnp.concatenate([dec_b1, dec_b1]).reshape(1, 2 * HID)
    bd2b = jnp.concatenate([dec_b2, dec_b2]).reshape(1, 2 * IN_DIM)
    bsb = jnp.concatenate([str_b, str_b]).reshape(1, 2 * HID)
    sel = (lax.broadcasted_iota(jnp.int32, (128, 8), 0)
           == 16 * lax.broadcasted_iota(jnp.int32, (128, 8), 1)).astype(f32)
    rep = (lax.broadcasted_iota(jnp.int32, (8, 512), 0)
           == lax.broadcasted_iota(jnp.int32, (8, 512), 1) // 64).astype(f32)

    degs = _sc_degree(dst_p)
    degs_p = degs.reshape(NC, N_PAD * DEGW // 128, 128)

    sd = lambda shape: jax.ShapeDtypeStruct(shape, f32)
    sp = lambda a: a.reshape(NC, NP2, 128)
    dinv, g1 = _full([sd((NP2, 128)), sd((NP2, 128))], _tc_pre_body)(
        x2, w1b, degs_p, sel, rep)
    s1 = _sc_prop(g1.reshape(N_PAD, HID), src_p, dst_p)
    g2 = _full(sd((NP2, 128)), _tc_enc_body)(sp(s1), g1, dinv, b1b, w2b)
    s2 = _sc_prop(g2.reshape(N_PAD, HID), src_p, dst_p)
    g3 = _full(sd((NP2, 128)), _tc_emb_body)(sp(s2), g2, dinv, b2b)
    s3 = _sc_prop(g3.reshape(N_PAD, HID), src_p, dst_p)
    g4, hs_p = _full([sd((NP2, 128)), sd((NP2, 128))], _tc_dec_body)(
        sp(s3), g3, dinv, bd1b, wd1b, bsb, wsb)
    s4 = _sc_prop(g4.reshape(N_PAD, HID), src_p, dst_p)
    x2o = _full(sd((NP2, 2 * IN_DIM)), _tc_out_body)(
        sp(s4), g4, dinv, bd2b, wd2b)

    x_ = x2o.reshape(N_PAD, IN_DIM)[:N]
    hs_n = hs_p.reshape(N_PAD, HID)[:N]
    s_ = _gram(hs_n, hs_n.T)
    return (x_, s_)
